# baseline (speedup 1.0000x reference)
import functools

import jax
import jax.numpy as jnp
from jax import lax
from jax.experimental import pallas as pl
from jax.experimental.pallas import tpu as pltpu
from jax.experimental.pallas import tpu_sc as plsc

B = 8
N = 2048
D = 128
K = 8
TQ = 512

CHUNKS = ((0, 1), (1, 3), (4, 3), (7, 1))

ROWS = B * N * K
RPB = N * K
NW = 32
CH = 128
QPC = CH // K


def _make_topk(b0, nb):
    def body(q_ref, xf_ref, idx_ref):
        q = q_ref[0]
        xf = xf_ref[0]
        sqq = jnp.sum(q * q, axis=1, keepdims=True)
        sqk = jnp.sum(xf * xf, axis=1)[None, :]
        dot = lax.dot_general(q, xf, (((1,), (1,)), ((), ())),
                              preferred_element_type=jnp.float32)
        key = (sqq + sqk) - 2.0 * dot
        b = b0 + pl.program_id(0)
        qi = pl.program_id(1)
        row_ids = qi * TQ + lax.broadcasted_iota(jnp.int32, (TQ, 1), 0)
        col_ids = lax.broadcasted_iota(jnp.int32, (TQ, N), 1)
        key = jnp.where(col_ids == row_ids, jnp.inf, key)
        idxs = [row_ids]
        for _ in range(K - 1):
            sel = jnp.argmin(key, axis=1).astype(jnp.int32)[:, None]
            idxs.append(sel)
            key = jnp.where(col_ids == sel, jnp.inf, key)
        idx_ref[0] = jnp.concatenate(idxs, axis=1) + b * N

    return pl.pallas_call(
        body,
        grid=(nb, N // TQ),
        in_specs=[
            pl.BlockSpec((1, TQ, D), lambda b, q: (b0 + b, q, 0)),
            pl.BlockSpec((1, N, D), lambda b, q: (b0 + b, 0, 0)),
        ],
        out_specs=pl.BlockSpec((1, TQ, K), lambda b, q: (b, q, 0)),
        out_shape=jax.ShapeDtypeStruct((nb, N, K), jnp.int32),
    )


def _sc_body(row0, nch, x_hbm, idx_hbm, out_hbm,
             idx_all, buf0, buf1, gs0, gs1, ws0, ws1):
    wid = lax.axis_index("s") * 2 + lax.axis_index("c")
    base = row0 + wid * (nch * CH)
    bufs = (buf0, buf1)
    gsems = (gs0, gs1)
    wsems = (ws0, ws1)

    pltpu.sync_copy(idx_hbm.at[wid], idx_all)

    ghandles = [None, None]
    whandles = [None, None]
    ghandles[0] = pltpu.async_copy(x_hbm.at[idx_all.at[0]], bufs[0], gsems[0])
    for g in range(nch):
        cur = g % 2
        nxt = g + 1
        if nxt < nch:
            if whandles[nxt % 2] is not None:
                whandles[nxt % 2].wait()
            ghandles[nxt % 2] = pltpu.async_copy(
                x_hbm.at[idx_all.at[nxt]], bufs[nxt % 2], gsems[nxt % 2])
        ghandles[cur].wait()
        buf = bufs[cur]

        def q_body(qq, carry):
            r0 = qq * K
            for j in range(D // 16):
                sl = pl.ds(j * 16, 16)
                h0 = buf[r0, sl] * 0.5
                for k in range(1, K):
                    buf[r0 + k, sl] = h0 + buf[r0 + k, sl] * 0.5
            return carry

        lax.fori_loop(0, QPC, q_body, 0)
        whandles[cur] = pltpu.async_copy(
            buf, out_hbm.at[pl.ds(base + g * CH, CH)], wsems[cur])
    if nch >= 2:
        whandles[(nch - 2) % 2].wait()
    whandles[(nch - 1) % 2].wait()


def _sc_scratch(nch):
    return [
        pltpu.VMEM((nch, CH), jnp.int32),
        pltpu.VMEM((CH, D), jnp.float32),
        pltpu.VMEM((CH, D), jnp.float32),
        pltpu.SemaphoreType.DMA,
        pltpu.SemaphoreType.DMA,
        pltpu.SemaphoreType.DMA,
        pltpu.SemaphoreType.DMA,
    ]


@functools.cache
def _make_sc(b0, nb, first):
    mesh = plsc.VectorSubcoreMesh(core_axis_name="c", subcore_axis_name="s")
    row0 = b0 * RPB
    nch = nb * RPB // NW // CH
    out_type = jax.ShapeDtypeStruct((ROWS, D), jnp.float32) if first else ()

    @functools.partial(
        pl.kernel, mesh=mesh, out_type=out_type,
        scratch_types=_sc_scratch(nch),
    )
    def _sc_expand(x_hbm, idx_hbm, out_hbm, *scratch):
        _sc_body(row0, nch, x_hbm, idx_hbm, out_hbm, *scratch)

    return _sc_expand


def kernel(x):
    xf = x.reshape(B * N, D)
    out_ref = None
    for ci, (b0, nb) in enumerate(CHUNKS):
        idx = _make_topk(b0, nb)(x, x).reshape(NW, nb * RPB // NW // CH, CH)
        if ci == 0:
            out0 = _make_sc(b0, nb, True)(xf, idx)
            out_ref = jax.new_ref(out0)
        else:
            _make_sc(b0, nb, False)(xf, idx, out_ref)
    return out_ref[...].reshape(B, N * K, D)


# speedup vs baseline: 1.0383x; 1.0383x over previous
import functools

import jax
import jax.numpy as jnp
from jax import lax
from jax.experimental import pallas as pl
from jax.experimental.pallas import tpu as pltpu
from jax.experimental.pallas import tpu_sc as plsc

B = 8
N = 2048
D = 128
K = 8
TQ = 512

CHUNKS = ((0, 2), (2, 2), (4, 2), (6, 2))

ROWS = B * N * K
RPB = N * K
NW = 32
CH = 128
QPC = CH // K


def _make_topk(b0, nb):
    def body(q_ref, xf_ref, idx_ref):
        q = q_ref[0]
        xf = xf_ref[0]
        sqq = jnp.sum(q * q, axis=1, keepdims=True)
        sqk = jnp.sum(xf * xf, axis=1)[None, :]
        dot = lax.dot_general(q, xf, (((1,), (1,)), ((), ())),
                              preferred_element_type=jnp.float32)
        key = (sqq + sqk) - 2.0 * dot
        b = b0 + pl.program_id(0)
        qi = pl.program_id(1)
        row_ids = qi * TQ + lax.broadcasted_iota(jnp.int32, (TQ, 1), 0)
        col_ids = lax.broadcasted_iota(jnp.int32, (TQ, N), 1)
        key = jnp.where(col_ids == row_ids, jnp.inf, key)
        idxs = [row_ids]
        for _ in range(K - 1):
            sel = jnp.argmin(key, axis=1).astype(jnp.int32)[:, None]
            idxs.append(sel)
            key = jnp.where(col_ids == sel, jnp.inf, key)
        idx_ref[0] = jnp.concatenate(idxs, axis=1) + b * N

    return pl.pallas_call(
        body,
        grid=(nb, N // TQ),
        in_specs=[
            pl.BlockSpec((1, TQ, D), lambda b, q: (b0 + b, q, 0)),
            pl.BlockSpec((1, N, D), lambda b, q: (b0 + b, 0, 0)),
        ],
        out_specs=pl.BlockSpec((1, TQ, K), lambda b, q: (b, q, 0)),
        out_shape=jax.ShapeDtypeStruct((nb, N, K), jnp.int32),
    )


def _sc_body(row0, nch, x_hbm, idx_hbm, out_hbm,
             idx_all, buf0, buf1, gs0, gs1, ws0, ws1):
    wid = lax.axis_index("s") * 2 + lax.axis_index("c")
    base = row0 + wid * (nch * CH)
    bufs = (buf0, buf1)
    gsems = (gs0, gs1)
    wsems = (ws0, ws1)

    pltpu.sync_copy(idx_hbm.at[wid], idx_all)

    ghandles = [None, None]
    whandles = [None, None]
    ghandles[0] = pltpu.async_copy(x_hbm.at[idx_all.at[0]], bufs[0], gsems[0])
    for g in range(nch):
        cur = g % 2
        nxt = g + 1
        if nxt < nch:
            if whandles[nxt % 2] is not None:
                whandles[nxt % 2].wait()
            ghandles[nxt % 2] = pltpu.async_copy(
                x_hbm.at[idx_all.at[nxt]], bufs[nxt % 2], gsems[nxt % 2])
        ghandles[cur].wait()
        buf = bufs[cur]

        def q_body(qq, carry):
            r0 = qq * K
            for j in range(D // 16):
                sl = pl.ds(j * 16, 16)
                h0 = buf[r0, sl] * 0.5
                for k in range(1, K):
                    buf[r0 + k, sl] = h0 + buf[r0 + k, sl] * 0.5
            return carry

        lax.fori_loop(0, QPC, q_body, 0)
        whandles[cur] = pltpu.async_copy(
            buf, out_hbm.at[pl.ds(base + g * CH, CH)], wsems[cur])
    if nch >= 2:
        whandles[(nch - 2) % 2].wait()
    whandles[(nch - 1) % 2].wait()


def _sc_scratch(nch):
    return [
        pltpu.VMEM((nch, CH), jnp.int32),
        pltpu.VMEM((CH, D), jnp.float32),
        pltpu.VMEM((CH, D), jnp.float32),
        pltpu.SemaphoreType.DMA,
        pltpu.SemaphoreType.DMA,
        pltpu.SemaphoreType.DMA,
        pltpu.SemaphoreType.DMA,
    ]


@functools.cache
def _make_sc(b0, nb, first):
    mesh = plsc.VectorSubcoreMesh(core_axis_name="c", subcore_axis_name="s")
    row0 = b0 * RPB
    nch = nb * RPB // NW // CH
    out_type = jax.ShapeDtypeStruct((ROWS, D), jnp.float32) if first else ()

    @functools.partial(
        pl.kernel, mesh=mesh, out_type=out_type,
        scratch_types=_sc_scratch(nch),
    )
    def _sc_expand(x_hbm, idx_hbm, out_hbm, *scratch):
        _sc_body(row0, nch, x_hbm, idx_hbm, out_hbm, *scratch)

    return _sc_expand


def kernel(x):
    xf = x.reshape(B * N, D)
    out_ref = None
    for ci, (b0, nb) in enumerate(CHUNKS):
        idx = _make_topk(b0, nb)(x, x).reshape(NW, nb * RPB // NW // CH, CH)
        if ci == 0:
            out0 = _make_sc(b0, nb, True)(xf, idx)
            out_ref = jax.new_ref(out0)
        else:
            _make_sc(b0, nb, False)(xf, idx, out_ref)
    return out_ref[...].reshape(B, N * K, D)
